# baseline (speedup 1.0000x reference)
import functools

import jax
import jax.numpy as jnp
from jax import lax
from jax.experimental import pallas as pl
from jax.experimental.pallas import tpu as pltpu
from jax.experimental.pallas import tpu_sc as plsc

_SC_CORES = 2
_SC_SUBCORES = 16


def _nn_kernel(ut_ref, kn_ref, wq_ref, gi_ref, *, M):
    ut = ut_ref[0]
    kn = kn_ref[0]
    u2 = jnp.sum(ut * ut, axis=0, keepdims=True)
    k2 = jnp.sum(kn * kn, axis=1, keepdims=True)
    d2 = u2 + k2 - 2.0 * jnp.dot(kn.astype(jnp.bfloat16), ut.astype(jnp.bfloat16),
                                 preferred_element_type=jnp.float32)
    miota = jax.lax.broadcasted_iota(jnp.int32, d2.shape, 0)
    cur = d2
    dists, idxs = [], []
    for _ in range(3):
        mn = jnp.min(cur, axis=0, keepdims=True)
        ik = jnp.min(jnp.where(cur == mn, miota, M), axis=0, keepdims=True)
        cur = jnp.where(miota == ik, jnp.float32(jnp.inf), cur)
        dists.append(jnp.maximum(mn, 0.0))
        idxs.append(ik)
    r = [1.0 / (d + 1e-8) for d in dists]
    norm = r[0] + r[1] + r[2]
    wq_ref[0] = jnp.concatenate([rk / norm for rk in r], axis=0)
    gi_ref[0] = jnp.concatenate(idxs, axis=0) + pl.program_id(0) * M


def _sc_interp_body(kft_hbm, gidx_hbm, w_hbm, out_hbm,
                    idx_v, rows_v, w_v, out_v, sem, *, N, C2, ppw, C):
    wid = lax.axis_index("s") * _SC_CORES + lax.axis_index("c")
    base = wid * ppw
    b = base // N
    n0 = base - b * N

    def chunk(ci, carry):
        off = b * (3 * N) + n0 + ci * C
        for k in range(3):
            pltpu.sync_copy(gidx_hbm.at[pl.ds(off + k * N, C)], idx_v.at[k])
            pltpu.sync_copy(w_hbm.at[pl.ds(off + k * N, C)], w_v.at[pl.ds(k * C, C)])
        cps = [pltpu.async_copy(kft_hbm.at[idx_v.at[k]], rows_v.at[k], sem)
               for k in range(3)]
        for cp in cps:
            cp.wait()

        def group(g, c2):
            p0 = g * 16
            wvec = [w_v[pl.ds(k * C + p0, 16)] for k in range(3)]
            for l in range(16):
                i = p0 + l
                lane = jnp.full((16, 1), l, jnp.int32)
                dn = lax.GatherDimensionNumbers(
                    offset_dims=(), collapsed_slice_dims=(0,), start_index_map=(0,))
                wks = [lax.gather(wvec[k], lane, dn, slice_sizes=(1,),
                                  mode=lax.GatherScatterMode.PROMISE_IN_BOUNDS)
                       for k in range(3)]
                for j in range(C2 // 16):
                    sl = pl.ds(j * 16, 16)
                    out_v[i, sl] = (wks[0] * rows_v[0, i, sl]
                                    + wks[1] * rows_v[1, i, sl]
                                    + wks[2] * rows_v[2, i, sl])
            return c2

        lax.fori_loop(0, C // 16, group, 0)
        pltpu.sync_copy(out_v, out_hbm.at[pl.ds(base + ci * C, C)])
        return carry

    lax.fori_loop(0, ppw // C, chunk, 0)


def _accum_stats(st_ref, y):
    @pl.when((pl.program_id(0) == 0) & (pl.program_id(1) == 0))
    def _():
        st_ref[...] = jnp.zeros_like(st_ref)

    st_ref[:, 0:1] += jnp.sum(y, axis=1, keepdims=True)
    st_ref[:, 1:2] += jnp.sum(y * y, axis=1, keepdims=True)


def _layer0_kernel(it_ref, uf_ref, w_ref, y_ref, st_ref, *, C2):
    it = it_ref[0].astype(jnp.bfloat16)
    y = lax.dot_general(w_ref[:, :C2], it, (((1,), (1,)), ((), ())),
                        preferred_element_type=jnp.float32)
    y += jnp.dot(w_ref[:, C2:], uf_ref[0].astype(jnp.bfloat16),
                 preferred_element_type=jnp.float32)
    y_ref[0] = y.astype(y_ref.dtype)
    _accum_stats(st_ref, y)


def _bn_scale_shift(st, g, b, count):
    mean = st[:, 0:1] * (1.0 / count)
    var = st[:, 1:2] * (1.0 / count) - mean * mean
    scale = g * jax.lax.rsqrt(var + 1e-5)
    return scale, b - mean * scale


def _bn_mm_kernel(x_ref, st0_ref, g_ref, b_ref, w_ref, y_ref, st_ref, *, count):
    scale, shift = _bn_scale_shift(st0_ref[...], g_ref[...], b_ref[...], count)
    z = jnp.maximum(x_ref[0].astype(jnp.float32) * scale + shift, 0.0)
    y = jnp.dot(w_ref[...], z.astype(jnp.bfloat16),
                preferred_element_type=jnp.float32)
    y_ref[0] = y.astype(y_ref.dtype)
    _accum_stats(st_ref, y)


def _bn_relu_kernel(x_ref, st0_ref, g_ref, b_ref, y_ref, *, count):
    scale, shift = _bn_scale_shift(st0_ref[...], g_ref[...], b_ref[...], count)
    y_ref[0] = jnp.maximum(x_ref[0].astype(jnp.float32) * scale + shift, 0.0)


def kernel(unknown, known, unknow_feats, known_feats,
           W0, gamma0, beta0, W1, gamma1, beta1, W2, gamma2, beta2):
    B, N, _ = unknown.shape
    M = known.shape[1]
    C1 = unknow_feats.shape[1]
    C2 = known_feats.shape[1]
    f32 = jnp.float32
    ut = jnp.transpose(unknown, (0, 2, 1))

    TN = min(4096, N)
    grid = (B, N // TN)
    wq, gidx = pl.pallas_call(
        functools.partial(_nn_kernel, M=M),
        grid=grid,
        in_specs=[
            pl.BlockSpec((1, 3, TN), lambda b, n: (b, 0, n)),
            pl.BlockSpec((1, M, 3), lambda b, n: (b, 0, 0)),
        ],
        out_specs=[pl.BlockSpec((1, 3, TN), lambda b, n: (b, 0, n)),
                   pl.BlockSpec((1, 3, TN), lambda b, n: (b, 0, n))],
        out_shape=[jax.ShapeDtypeStruct((B, 3, N), f32),
                   jax.ShapeDtypeStruct((B, 3, N), jnp.int32)],
    )(ut, known)

    kft = jnp.transpose(known_feats, (0, 2, 1)).reshape(B * M, C2)
    nw = _SC_CORES * _SC_SUBCORES
    ppw = (B * N) // nw
    C = 64
    sc_call = pl.kernel(
        functools.partial(_sc_interp_body, N=N, C2=C2, ppw=ppw, C=C),
        mesh=plsc.VectorSubcoreMesh(core_axis_name="c", subcore_axis_name="s"),
        out_type=jax.ShapeDtypeStruct((B * N, C2), f32),
        scratch_types=[
            pltpu.VMEM((3, C), jnp.int32),
            pltpu.VMEM((3, C, C2), f32),
            pltpu.VMEM((3 * C,), f32),
            pltpu.VMEM((C, C2), f32),
            pltpu.SemaphoreType.DMA,
        ],
    )
    interp = sc_call(kft, gidx.reshape(B * 3 * N), wq.reshape(B * 3 * N))
    interp = interp.reshape(B, N, C2)

    count = B * N
    O0, O1, O2 = W0.shape[0], W1.shape[0], W2.shape[0]

    def cm_spec(Cc):
        return pl.BlockSpec((1, Cc, TN), lambda b, n: (b, 0, n))

    def full2d(a):
        return pl.BlockSpec(a.shape, lambda b, n: (0, 0))

    y0, st0 = pl.pallas_call(
        functools.partial(_layer0_kernel, C2=C2),
        grid=grid,
        in_specs=[pl.BlockSpec((1, TN, C2), lambda b, n: (b, n, 0)),
                  cm_spec(C1), full2d(W0)],
        out_specs=[cm_spec(O0), pl.BlockSpec((O0, 2), lambda b, n: (0, 0))],
        out_shape=[jax.ShapeDtypeStruct((B, O0, N), jnp.bfloat16),
                   jax.ShapeDtypeStruct((O0, 2), f32)],
    )(interp, unknow_feats, W0.astype(jnp.bfloat16))

    def bn_layer(x, st, g, b, W, Oin, Oout):
        return pl.pallas_call(
            functools.partial(_bn_mm_kernel, count=count),
            grid=grid,
            in_specs=[cm_spec(Oin), pl.BlockSpec((Oin, 2), lambda b, n: (0, 0)),
                      pl.BlockSpec((Oin, 1), lambda b, n: (0, 0)),
                      pl.BlockSpec((Oin, 1), lambda b, n: (0, 0)), full2d(W)],
            out_specs=[cm_spec(Oout), pl.BlockSpec((Oout, 2), lambda b, n: (0, 0))],
            out_shape=[jax.ShapeDtypeStruct((B, Oout, N), jnp.bfloat16),
                       jax.ShapeDtypeStruct((Oout, 2), f32)],
        )(x, st, g.reshape(Oin, 1), b.reshape(Oin, 1), W.astype(jnp.bfloat16))

    y1, st1 = bn_layer(y0, st0, gamma0, beta0, W1, O0, O1)
    y2, st2 = bn_layer(y1, st1, gamma1, beta1, W2, O1, O2)

    out = pl.pallas_call(
        functools.partial(_bn_relu_kernel, count=count),
        grid=grid,
        in_specs=[cm_spec(O2), pl.BlockSpec((O2, 2), lambda b, n: (0, 0)),
                  pl.BlockSpec((O2, 1), lambda b, n: (0, 0)),
                  pl.BlockSpec((O2, 1), lambda b, n: (0, 0))],
        out_specs=cm_spec(O2),
        out_shape=jax.ShapeDtypeStruct((B, O2, N), f32),
    )(y2, st2, gamma2.reshape(O2, 1), beta2.reshape(O2, 1))
    return out


# speedup vs baseline: 1.0684x; 1.0684x over previous
import functools

import jax
import jax.numpy as jnp
from jax import lax
from jax.experimental import pallas as pl
from jax.experimental.pallas import tpu as pltpu
from jax.experimental.pallas import tpu_sc as plsc

_SC_CORES = 2
_SC_SUBCORES = 16


def _nn_kernel(ut_ref, kn_ref, wq_ref, gi_ref, *, M):
    ut = ut_ref[0]
    kn = kn_ref[0]
    u2 = jnp.sum(ut * ut, axis=0, keepdims=True)
    k2 = jnp.sum(kn * kn, axis=1, keepdims=True)
    d2 = u2 + k2 - 2.0 * jnp.dot(kn.astype(jnp.bfloat16), ut.astype(jnp.bfloat16),
                                 preferred_element_type=jnp.float32)
    miota = jax.lax.broadcasted_iota(jnp.int32, d2.shape, 0)
    cur = d2
    dists, idxs = [], []
    for _ in range(3):
        mn = jnp.min(cur, axis=0, keepdims=True)
        ik = jnp.min(jnp.where(cur == mn, miota, M), axis=0, keepdims=True)
        cur = jnp.where(miota == ik, jnp.float32(jnp.inf), cur)
        dists.append(jnp.maximum(mn, 0.0))
        idxs.append(ik)
    r = [1.0 / (d + 1e-8) for d in dists]
    norm = r[0] + r[1] + r[2]
    wq_ref[0] = jnp.concatenate([rk / norm for rk in r], axis=0)
    gi_ref[0] = jnp.concatenate(idxs, axis=0) + pl.program_id(0) * M


def _sc_interp_body(kft_hbm, gidx_hbm, w_hbm, out_hbm,
                    idx_v, rows_v, w_v, out_v, sem0, sem1, *, N, C2, ppw, C):
    wid = lax.axis_index("s") * _SC_CORES + lax.axis_index("c")
    base = wid * ppw
    b = base // N
    n0 = base - b * N
    nch = ppw // C
    sems = (sem0, sem1)

    pltpu.sync_copy(gidx_hbm.at[b, :, pl.ds(n0, ppw)], idx_v)
    pltpu.sync_copy(w_hbm.at[b, :, pl.ds(n0, ppw)], w_v)

    def issue(ci, buf):
        for k in range(3):
            pltpu.async_copy(kft_hbm.at[idx_v.at[k, pl.ds(ci * C, C)]],
                             rows_v.at[buf, k], sems[buf])

    def drain(buf):
        for k in range(3):
            pltpu.make_async_copy(kft_hbm.at[idx_v.at[k, pl.ds(0, C)]],
                                  rows_v.at[buf, k], sems[buf]).wait()

    def combine(ci, buf):
        def group(g, c):
            p0 = g * 16
            wvec = [w_v[k, pl.ds(ci * C + p0, 16)] for k in range(3)]
            for l in range(16):
                i = p0 + l
                lane = jnp.full((16, 1), l, jnp.int32)
                dn = lax.GatherDimensionNumbers(
                    offset_dims=(), collapsed_slice_dims=(0,), start_index_map=(0,))
                wks = [lax.gather(wvec[k], lane, dn, slice_sizes=(1,),
                                  mode=lax.GatherScatterMode.PROMISE_IN_BOUNDS)
                       for k in range(3)]
                for j in range(C2 // 16):
                    sl = pl.ds(j * 16, 16)
                    out_v[i, sl] = (wks[0] * rows_v[buf, 0, i, sl]
                                    + wks[1] * rows_v[buf, 1, i, sl]
                                    + wks[2] * rows_v[buf, 2, i, sl])
            return c

        lax.fori_loop(0, C // 16, group, 0)
        pltpu.sync_copy(out_v, out_hbm.at[pl.ds(base + ci * C, C)])

    issue(0, 0)

    def pair(p, carry):
        ci = 2 * p
        issue(ci + 1, 1)
        drain(0)
        combine(ci, 0)

        @pl.when(ci + 2 < nch)
        def _():
            issue(ci + 2, 0)

        drain(1)
        combine(ci + 1, 1)
        return carry

    lax.fori_loop(0, nch // 2, pair, 0)


def _accum_stats(st_ref, y):
    @pl.when((pl.program_id(0) == 0) & (pl.program_id(1) == 0))
    def _():
        st_ref[...] = jnp.zeros_like(st_ref)

    st_ref[:, 0:1] += jnp.sum(y, axis=1, keepdims=True)
    st_ref[:, 1:2] += jnp.sum(y * y, axis=1, keepdims=True)


def _layer0_kernel(it_ref, uf_ref, w_ref, y_ref, st_ref, *, C2):
    it = it_ref[0].astype(jnp.bfloat16)
    y = lax.dot_general(w_ref[:, :C2], it, (((1,), (1,)), ((), ())),
                        preferred_element_type=jnp.float32)
    y += jnp.dot(w_ref[:, C2:], uf_ref[0].astype(jnp.bfloat16),
                 preferred_element_type=jnp.float32)
    y_ref[0] = y.astype(y_ref.dtype)
    _accum_stats(st_ref, y)


def _bn_scale_shift(st, g, b, count):
    mean = st[:, 0:1] * (1.0 / count)
    var = st[:, 1:2] * (1.0 / count) - mean * mean
    scale = g * jax.lax.rsqrt(var + 1e-5)
    return scale, b - mean * scale


def _bn_mm_kernel(x_ref, st0_ref, g_ref, b_ref, w_ref, y_ref, st_ref, *, count):
    scale, shift = _bn_scale_shift(st0_ref[...], g_ref[...], b_ref[...], count)
    z = jnp.maximum(x_ref[0].astype(jnp.float32) * scale + shift, 0.0)
    y = jnp.dot(w_ref[...], z.astype(jnp.bfloat16),
                preferred_element_type=jnp.float32)
    y_ref[0] = y.astype(y_ref.dtype)
    _accum_stats(st_ref, y)


def _bn_relu_kernel(x_ref, st0_ref, g_ref, b_ref, y_ref, *, count):
    scale, shift = _bn_scale_shift(st0_ref[...], g_ref[...], b_ref[...], count)
    y_ref[0] = jnp.maximum(x_ref[0].astype(jnp.float32) * scale + shift, 0.0)


def kernel(unknown, known, unknow_feats, known_feats,
           W0, gamma0, beta0, W1, gamma1, beta1, W2, gamma2, beta2):
    B, N, _ = unknown.shape
    M = known.shape[1]
    C1 = unknow_feats.shape[1]
    C2 = known_feats.shape[1]
    f32 = jnp.float32
    ut = jnp.transpose(unknown, (0, 2, 1))

    TN = min(4096, N)
    grid = (B, N // TN)
    wq, gidx = pl.pallas_call(
        functools.partial(_nn_kernel, M=M),
        grid=grid,
        in_specs=[
            pl.BlockSpec((1, 3, TN), lambda b, n: (b, 0, n)),
            pl.BlockSpec((1, M, 3), lambda b, n: (b, 0, 0)),
        ],
        out_specs=[pl.BlockSpec((1, 3, TN), lambda b, n: (b, 0, n)),
                   pl.BlockSpec((1, 3, TN), lambda b, n: (b, 0, n))],
        out_shape=[jax.ShapeDtypeStruct((B, 3, N), f32),
                   jax.ShapeDtypeStruct((B, 3, N), jnp.int32)],
    )(ut, known)

    kft = jnp.transpose(known_feats, (0, 2, 1)).reshape(B * M, C2)
    nw = _SC_CORES * _SC_SUBCORES
    ppw = (B * N) // nw
    C = 32
    sc_call = pl.kernel(
        functools.partial(_sc_interp_body, N=N, C2=C2, ppw=ppw, C=C),
        mesh=plsc.VectorSubcoreMesh(core_axis_name="c", subcore_axis_name="s"),
        out_type=jax.ShapeDtypeStruct((B * N, C2), f32),
        scratch_types=[
            pltpu.VMEM((3, ppw), jnp.int32),
            pltpu.VMEM((2, 3, C, C2), f32),
            pltpu.VMEM((3, ppw), f32),
            pltpu.VMEM((C, C2), f32),
            pltpu.SemaphoreType.DMA,
            pltpu.SemaphoreType.DMA,
        ],
    )
    interp = sc_call(kft, gidx, wq)
    interp = interp.reshape(B, N, C2)

    count = B * N
    O0, O1, O2 = W0.shape[0], W1.shape[0], W2.shape[0]

    def cm_spec(Cc):
        return pl.BlockSpec((1, Cc, TN), lambda b, n: (b, 0, n))

    def full2d(a):
        return pl.BlockSpec(a.shape, lambda b, n: (0, 0))

    y0, st0 = pl.pallas_call(
        functools.partial(_layer0_kernel, C2=C2),
        grid=grid,
        in_specs=[pl.BlockSpec((1, TN, C2), lambda b, n: (b, n, 0)),
                  cm_spec(C1), full2d(W0)],
        out_specs=[cm_spec(O0), pl.BlockSpec((O0, 2), lambda b, n: (0, 0))],
        out_shape=[jax.ShapeDtypeStruct((B, O0, N), jnp.bfloat16),
                   jax.ShapeDtypeStruct((O0, 2), f32)],
    )(interp, unknow_feats, W0.astype(jnp.bfloat16))

    def bn_layer(x, st, g, b, W, Oin, Oout):
        return pl.pallas_call(
            functools.partial(_bn_mm_kernel, count=count),
            grid=grid,
            in_specs=[cm_spec(Oin), pl.BlockSpec((Oin, 2), lambda b, n: (0, 0)),
                      pl.BlockSpec((Oin, 1), lambda b, n: (0, 0)),
                      pl.BlockSpec((Oin, 1), lambda b, n: (0, 0)), full2d(W)],
            out_specs=[cm_spec(Oout), pl.BlockSpec((Oout, 2), lambda b, n: (0, 0))],
            out_shape=[jax.ShapeDtypeStruct((B, Oout, N), jnp.bfloat16),
                       jax.ShapeDtypeStruct((Oout, 2), f32)],
        )(x, st, g.reshape(Oin, 1), b.reshape(Oin, 1), W.astype(jnp.bfloat16))

    y1, st1 = bn_layer(y0, st0, gamma0, beta0, W1, O0, O1)
    y2, st2 = bn_layer(y1, st1, gamma1, beta1, W2, O1, O2)

    out = pl.pallas_call(
        functools.partial(_bn_relu_kernel, count=count),
        grid=grid,
        in_specs=[cm_spec(O2), pl.BlockSpec((O2, 2), lambda b, n: (0, 0)),
                  pl.BlockSpec((O2, 1), lambda b, n: (0, 0)),
                  pl.BlockSpec((O2, 1), lambda b, n: (0, 0))],
        out_specs=cm_spec(O2),
        out_shape=jax.ShapeDtypeStruct((B, O2, N), f32),
    )(y2, st2, gamma2.reshape(O2, 1), beta2.reshape(O2, 1))
    return out


# speedup vs baseline: 1.0845x; 1.0150x over previous
import functools

import jax
import jax.numpy as jnp
from jax import lax
from jax.experimental import pallas as pl
from jax.experimental.pallas import tpu as pltpu
from jax.experimental.pallas import tpu_sc as plsc

_SC_CORES = 2
_SC_SUBCORES = 16


def _nn_kernel(ut_ref, kn_ref, wq_ref, gi_ref, *, M):
    ut = ut_ref[0]
    kn = kn_ref[0]
    u2 = jnp.sum(ut * ut, axis=0, keepdims=True)
    k2 = jnp.sum(kn * kn, axis=1, keepdims=True)
    d2 = u2 + k2 - 2.0 * jnp.dot(kn.astype(jnp.bfloat16), ut.astype(jnp.bfloat16),
                                 preferred_element_type=jnp.float32)
    miota = jax.lax.broadcasted_iota(jnp.int32, d2.shape, 0)
    cur = d2
    dists, idxs = [], []
    for _ in range(3):
        mn = jnp.min(cur, axis=0, keepdims=True)
        ik = jnp.min(jnp.where(cur == mn, miota, M), axis=0, keepdims=True)
        cur = jnp.where(miota == ik, jnp.float32(jnp.inf), cur)
        dists.append(jnp.maximum(mn, 0.0))
        idxs.append(ik)
    r = [1.0 / (d + 1e-8) for d in dists]
    norm = r[0] + r[1] + r[2]
    wq_ref[0] = jnp.concatenate([rk / norm for rk in r], axis=0)
    gi_ref[0] = jnp.concatenate(idxs, axis=0) + pl.program_id(0) * M


def _sc_interp_body(kft_hbm, gidx_hbm, w_hbm, out_hbm,
                    idx_v, rows_v, w_v, out_v, sem0, sem1, so0, so1,
                    *, N, C2, ppw, C):
    wid = lax.axis_index("s") * _SC_CORES + lax.axis_index("c")
    base = wid * ppw
    b = base // N
    n0 = base - b * N
    nch = ppw // C
    sems = (sem0, sem1)
    souts = (so0, so1)

    pltpu.sync_copy(gidx_hbm.at[b, :, pl.ds(n0, ppw)], idx_v)
    pltpu.sync_copy(w_hbm.at[b, :, pl.ds(n0, ppw)], w_v)

    def issue(ci, buf):
        for k in range(3):
            pltpu.async_copy(kft_hbm.at[idx_v.at[k, pl.ds(ci * C, C)]],
                             rows_v.at[buf, k], sems[buf])

    def drain(buf):
        for k in range(3):
            pltpu.make_async_copy(kft_hbm.at[idx_v.at[k, pl.ds(0, C)]],
                                  rows_v.at[buf, k], sems[buf]).wait()

    def combine(ci, buf):
        @pl.when(ci >= 2)
        def _():
            pltpu.make_async_copy(out_v.at[buf], out_hbm.at[pl.ds(0, C)],
                                  souts[buf]).wait()

        def group(g, c):
            p0 = g * 16
            wvec = [w_v[k, pl.ds(ci * C + p0, 16)] for k in range(3)]
            for l in range(16):
                i = p0 + l
                lane = jnp.full((16, 1), l, jnp.int32)
                dn = lax.GatherDimensionNumbers(
                    offset_dims=(), collapsed_slice_dims=(0,), start_index_map=(0,))
                wks = [lax.gather(wvec[k], lane, dn, slice_sizes=(1,),
                                  mode=lax.GatherScatterMode.PROMISE_IN_BOUNDS)
                       for k in range(3)]
                for j in range(C2 // 16):
                    sl = pl.ds(j * 16, 16)
                    out_v[buf, i, sl] = (wks[0] * rows_v[buf, 0, i, sl]
                                    + wks[1] * rows_v[buf, 1, i, sl]
                                    + wks[2] * rows_v[buf, 2, i, sl])
            return c

        lax.fori_loop(0, C // 16, group, 0)
        pltpu.async_copy(out_v.at[buf], out_hbm.at[pl.ds(base + ci * C, C)],
                         souts[buf])

    issue(0, 0)

    def pair(p, carry):
        ci = 2 * p
        issue(ci + 1, 1)
        drain(0)
        combine(ci, 0)

        @pl.when(ci + 2 < nch)
        def _():
            issue(ci + 2, 0)

        drain(1)
        combine(ci + 1, 1)
        return carry

    lax.fori_loop(0, nch // 2, pair, 0)
    for buf in (0, 1):
        pltpu.make_async_copy(out_v.at[buf], out_hbm.at[pl.ds(0, C)],
                              souts[buf]).wait()


def _accum_stats(st_ref, y):
    @pl.when((pl.program_id(0) == 0) & (pl.program_id(1) == 0))
    def _():
        st_ref[...] = jnp.zeros_like(st_ref)

    st_ref[:, 0:1] += jnp.sum(y, axis=1, keepdims=True)
    st_ref[:, 1:2] += jnp.sum(y * y, axis=1, keepdims=True)


def _layer0_kernel(it_ref, uf_ref, w_ref, y_ref, st_ref, *, C2):
    it = it_ref[0].astype(jnp.bfloat16)
    y = lax.dot_general(w_ref[:, :C2], it, (((1,), (1,)), ((), ())),
                        preferred_element_type=jnp.float32)
    y += jnp.dot(w_ref[:, C2:], uf_ref[0].astype(jnp.bfloat16),
                 preferred_element_type=jnp.float32)
    y_ref[0] = y.astype(y_ref.dtype)
    _accum_stats(st_ref, y)


def _bn_scale_shift(st, g, b, count):
    mean = st[:, 0:1] * (1.0 / count)
    var = st[:, 1:2] * (1.0 / count) - mean * mean
    scale = g * jax.lax.rsqrt(var + 1e-5)
    return scale, b - mean * scale


def _bn_mm_kernel(x_ref, st0_ref, g_ref, b_ref, w_ref, y_ref, st_ref, *, count):
    scale, shift = _bn_scale_shift(st0_ref[...], g_ref[...], b_ref[...], count)
    z = jnp.maximum(x_ref[0].astype(jnp.float32) * scale + shift, 0.0)
    y = jnp.dot(w_ref[...], z.astype(jnp.bfloat16),
                preferred_element_type=jnp.float32)
    y_ref[0] = y.astype(y_ref.dtype)
    _accum_stats(st_ref, y)


def _bn_relu_kernel(x_ref, st0_ref, g_ref, b_ref, y_ref, *, count):
    scale, shift = _bn_scale_shift(st0_ref[...], g_ref[...], b_ref[...], count)
    y_ref[0] = jnp.maximum(x_ref[0].astype(jnp.float32) * scale + shift, 0.0)


def kernel(unknown, known, unknow_feats, known_feats,
           W0, gamma0, beta0, W1, gamma1, beta1, W2, gamma2, beta2):
    B, N, _ = unknown.shape
    M = known.shape[1]
    C1 = unknow_feats.shape[1]
    C2 = known_feats.shape[1]
    f32 = jnp.float32
    ut = jnp.transpose(unknown, (0, 2, 1))

    TN = min(4096, N)
    grid = (B, N // TN)
    wq, gidx = pl.pallas_call(
        functools.partial(_nn_kernel, M=M),
        grid=grid,
        in_specs=[
            pl.BlockSpec((1, 3, TN), lambda b, n: (b, 0, n)),
            pl.BlockSpec((1, M, 3), lambda b, n: (b, 0, 0)),
        ],
        out_specs=[pl.BlockSpec((1, 3, TN), lambda b, n: (b, 0, n)),
                   pl.BlockSpec((1, 3, TN), lambda b, n: (b, 0, n))],
        out_shape=[jax.ShapeDtypeStruct((B, 3, N), f32),
                   jax.ShapeDtypeStruct((B, 3, N), jnp.int32)],
    )(ut, known)

    kft = jnp.transpose(known_feats, (0, 2, 1)).reshape(B * M, C2)
    nw = _SC_CORES * _SC_SUBCORES
    ppw = (B * N) // nw
    C = 32
    sc_call = pl.kernel(
        functools.partial(_sc_interp_body, N=N, C2=C2, ppw=ppw, C=C),
        mesh=plsc.VectorSubcoreMesh(core_axis_name="c", subcore_axis_name="s"),
        out_type=jax.ShapeDtypeStruct((B * N, C2), f32),
        scratch_types=[
            pltpu.VMEM((3, ppw), jnp.int32),
            pltpu.VMEM((2, 3, C, C2), f32),
            pltpu.VMEM((3, ppw), f32),
            pltpu.VMEM((2, C, C2), f32),
            pltpu.SemaphoreType.DMA,
            pltpu.SemaphoreType.DMA,
            pltpu.SemaphoreType.DMA,
            pltpu.SemaphoreType.DMA,
        ],
    )
    interp = sc_call(kft, gidx, wq)
    interp = interp.reshape(B, N, C2)

    count = B * N
    O0, O1, O2 = W0.shape[0], W1.shape[0], W2.shape[0]

    def cm_spec(Cc):
        return pl.BlockSpec((1, Cc, TN), lambda b, n: (b, 0, n))

    def full2d(a):
        return pl.BlockSpec(a.shape, lambda b, n: (0, 0))

    y0, st0 = pl.pallas_call(
        functools.partial(_layer0_kernel, C2=C2),
        grid=grid,
        in_specs=[pl.BlockSpec((1, TN, C2), lambda b, n: (b, n, 0)),
                  cm_spec(C1), full2d(W0)],
        out_specs=[cm_spec(O0), pl.BlockSpec((O0, 2), lambda b, n: (0, 0))],
        out_shape=[jax.ShapeDtypeStruct((B, O0, N), jnp.bfloat16),
                   jax.ShapeDtypeStruct((O0, 2), f32)],
    )(interp, unknow_feats, W0.astype(jnp.bfloat16))

    def bn_layer(x, st, g, b, W, Oin, Oout):
        return pl.pallas_call(
            functools.partial(_bn_mm_kernel, count=count),
            grid=grid,
            in_specs=[cm_spec(Oin), pl.BlockSpec((Oin, 2), lambda b, n: (0, 0)),
                      pl.BlockSpec((Oin, 1), lambda b, n: (0, 0)),
                      pl.BlockSpec((Oin, 1), lambda b, n: (0, 0)), full2d(W)],
            out_specs=[cm_spec(Oout), pl.BlockSpec((Oout, 2), lambda b, n: (0, 0))],
            out_shape=[jax.ShapeDtypeStruct((B, Oout, N), jnp.bfloat16),
                       jax.ShapeDtypeStruct((Oout, 2), f32)],
        )(x, st, g.reshape(Oin, 1), b.reshape(Oin, 1), W.astype(jnp.bfloat16))

    y1, st1 = bn_layer(y0, st0, gamma0, beta0, W1, O0, O1)
    y2, st2 = bn_layer(y1, st1, gamma1, beta1, W2, O1, O2)

    out = pl.pallas_call(
        functools.partial(_bn_relu_kernel, count=count),
        grid=grid,
        in_specs=[cm_spec(O2), pl.BlockSpec((O2, 2), lambda b, n: (0, 0)),
                  pl.BlockSpec((O2, 1), lambda b, n: (0, 0)),
                  pl.BlockSpec((O2, 1), lambda b, n: (0, 0))],
        out_specs=cm_spec(O2),
        out_shape=jax.ShapeDtypeStruct((B, O2, N), f32),
    )(y2, st2, gamma2.reshape(O2, 1), beta2.reshape(O2, 1))
    return out
